# baseline (speedup 1.0000x reference)
import functools

import jax
import jax.numpy as jnp
from jax import lax
from jax.experimental import pallas as pl
from jax.experimental.pallas import tpu as pltpu
from jax.experimental.pallas import tpu_sc as plsc

NUM_ROWS = 100000
DIM = 128
NUM_IDX = 50000
EPS = 1e-8

SC_TILES = 16
SC_CHUNKS = 25
SC_LANEBLK = 128
PAD_IDX = SC_TILES * SC_CHUNKS * SC_LANEBLK

BLK = 5000
NUM_BLKS = NUM_ROWS // BLK


def _sc_histogram(idx3, val3, zeros_hbm):
  mesh = plsc.VectorSubcoreMesh(core_axis_name="c", subcore_axis_name="s")

  @functools.partial(
      pl.kernel,
      mesh=mesh,
      out_type=[
          jax.ShapeDtypeStruct((NUM_ROWS,), jnp.float32),
          jax.ShapeDtypeStruct((NUM_ROWS,), jnp.float32),
      ],
      scratch_types=[
          pltpu.VMEM((SC_CHUNKS, SC_LANEBLK), jnp.int32),
          pltpu.VMEM((SC_CHUNKS, SC_LANEBLK), jnp.float32),
          pltpu.VMEM_SHARED((NUM_ROWS,), jnp.float32),
      ],
  )
  def k(idx_hbm, val_hbm, zero_hbm, out_p, out_n, idx_v, val_v, shared):
    c = lax.axis_index("c")
    s = lax.axis_index("s")

    @pl.when(s == 0)
    def _():
      pltpu.sync_copy(zero_hbm, shared)

    pltpu.sync_copy(idx_hbm.at[c, s], idx_v)
    pltpu.sync_copy(val_hbm.at[s], val_v)
    plsc.subcore_barrier()

    for j in range(SC_CHUNKS):
      pltpu.sync_copy(val_v.at[j], shared.at[idx_v.at[j]], add=True)

    plsc.subcore_barrier()

    @pl.when((s == 0) & (c == 0))
    def _():
      pltpu.sync_copy(shared, out_p)

    @pl.when((s == 0) & (c == 1))
    def _():
      pltpu.sync_copy(shared, out_n)

  return k(idx3, val3, zeros_hbm)


def _tc_norm_body(emb_ref, out_ref):
  e = emb_ref[...]
  eb = e.astype(jnp.bfloat16)
  sqb = eb * eb
  ones8 = jnp.ones((8, DIM), jnp.bfloat16)
  n28 = lax.dot_general(ones8, sqb, (((1,), (1,)), ((), ())),
                        preferred_element_type=jnp.float32)
  m = lax.rsqrt(jnp.maximum(n28[0:1], EPS * EPS))
  out_ref[...] = m[None]


def _tc_norm(embeddings):
  return pl.pallas_call(
      _tc_norm_body,
      grid=(NUM_BLKS,),
      in_specs=[pl.BlockSpec((BLK, DIM), lambda i: (i, 0))],
      out_specs=pl.BlockSpec((1, 1, BLK), lambda i: (i, 0, 0)),
      out_shape=jax.ShapeDtypeStruct((NUM_BLKS, 1, BLK), jnp.float32),
  )(embeddings)


def _tc_loss_body(emb_ref, r_ref, cp_ref, cn_ref, out_ref, acc_ref):
  i = pl.program_id(0)
  e = emb_ref[...]
  eb = e.astype(jnp.bfloat16)
  m = r_ref[0]
  cp = cp_ref[0]
  cn = cn_ref[0]
  w = jnp.concatenate(
      [cp, cn, cp * m, cn * m,
       jnp.zeros((4, BLK), jnp.float32)], axis=0)
  part = jnp.dot(w.astype(jnp.bfloat16), eb,
                 preferred_element_type=jnp.float32)

  @pl.when(i == 0)
  def _():
    acc_ref[...] = jnp.zeros_like(acc_ref)

  acc_ref[...] += part

  @pl.when(i == NUM_BLKS - 1)
  def _():
    a = acc_ref[...]
    n = jnp.float32(NUM_IDX)
    sp, sn, tp, tn = a[0], a[1], a[2], a[3]
    mcp = jnp.sum(sp * tp) / (n * jnp.maximum(jnp.sqrt(jnp.sum(sp * sp)),
                                              n * EPS))
    mcn = jnp.sum(sn * tn) / (n * jnp.maximum(jnp.sqrt(jnp.sum(sn * sn)),
                                              n * EPS))
    out_ref[...] = jnp.full((1, 1), 2.0 - mcp - mcn, jnp.float32)


def _tc_loss(embeddings, r3, cp3, cn3):
  return pl.pallas_call(
      _tc_loss_body,
      grid=(NUM_BLKS,),
      in_specs=[
          pl.BlockSpec((BLK, DIM), lambda i: (i, 0)),
          pl.BlockSpec((1, 1, BLK), lambda i: (i, 0, 0)),
          pl.BlockSpec((1, 1, BLK), lambda i: (i, 0, 0)),
          pl.BlockSpec((1, 1, BLK), lambda i: (i, 0, 0)),
      ],
      out_specs=pl.BlockSpec((1, 1), lambda i: (0, 0)),
      out_shape=jax.ShapeDtypeStruct((1, 1), jnp.float32),
      scratch_shapes=[pltpu.VMEM((8, DIM), jnp.float32)],
  )(embeddings, r3, cp3, cn3)


def kernel(embeddings, positive_nodes, negative_nodes):
  pad = PAD_IDX - NUM_IDX
  idx_p = jnp.concatenate(
      [positive_nodes.astype(jnp.int32),
       jnp.zeros((pad,), jnp.int32)]).reshape(SC_TILES, SC_CHUNKS, SC_LANEBLK)
  idx_n = jnp.concatenate(
      [negative_nodes.astype(jnp.int32),
       jnp.zeros((pad,), jnp.int32)]).reshape(SC_TILES, SC_CHUNKS, SC_LANEBLK)
  idx3 = jnp.stack([idx_p, idx_n], axis=0)
  val3 = jnp.concatenate(
      [jnp.ones((NUM_IDX,), jnp.float32),
       jnp.zeros((pad,), jnp.float32)]).reshape(SC_TILES, SC_CHUNKS,
                                                SC_LANEBLK)
  zeros_hbm = jnp.zeros((NUM_ROWS,), jnp.float32)

  r3 = _tc_norm(embeddings)
  hp, hn = _sc_histogram(idx3, val3, zeros_hbm)
  cp3 = hp.reshape(NUM_BLKS, 1, BLK)
  cn3 = hn.reshape(NUM_BLKS, 1, BLK)
  loss = _tc_loss(embeddings, r3, cp3, cn3)
  return loss[0, 0]


# speedup vs baseline: 1.2455x; 1.2455x over previous
import functools

import jax
import jax.numpy as jnp
from jax import lax
from jax.experimental import pallas as pl
from jax.experimental.pallas import tpu as pltpu
from jax.experimental.pallas import tpu_sc as plsc

NUM_ROWS = 100000
DIM = 128
NUM_IDX = 50000
EPS = 1e-8

SC_TILES = 16
SC_CHUNKS = 25
SC_LANEBLK = 128
PAD_IDX = SC_TILES * SC_CHUNKS * SC_LANEBLK

BLK = 5000
NUM_BLKS = NUM_ROWS // BLK


def _sc_histogram(idx3, val3, zeros_hbm):
  mesh = plsc.VectorSubcoreMesh(core_axis_name="c", subcore_axis_name="s")

  @functools.partial(
      pl.kernel,
      mesh=mesh,
      out_type=[
          jax.ShapeDtypeStruct((NUM_ROWS,), jnp.float32),
          jax.ShapeDtypeStruct((NUM_ROWS,), jnp.float32),
      ],
      scratch_types=[
          pltpu.VMEM((SC_CHUNKS, SC_LANEBLK), jnp.int32),
          pltpu.VMEM((SC_CHUNKS, SC_LANEBLK), jnp.float32),
          pltpu.VMEM_SHARED((NUM_ROWS,), jnp.float32),
          pltpu.SemaphoreType.DMA,
      ],
  )
  def k(idx_hbm, val_hbm, zero_hbm, out_p, out_n, idx_v, val_v, shared, sem):
    c = lax.axis_index("c")
    s = lax.axis_index("s")

    @pl.when(s == 0)
    def _():
      pltpu.sync_copy(zero_hbm, shared)

    pltpu.sync_copy(idx_hbm.at[c, s], idx_v)
    pltpu.sync_copy(val_hbm.at[s], val_v)
    plsc.subcore_barrier()

    copies = [
        pltpu.async_copy(val_v.at[j], shared.at[idx_v.at[j]], sem, add=True)
        for j in range(SC_CHUNKS)
    ]
    for cpy in copies:
      cpy.wait()

    plsc.subcore_barrier()

    @pl.when((s == 0) & (c == 0))
    def _():
      pltpu.sync_copy(shared, out_p)

    @pl.when((s == 0) & (c == 1))
    def _():
      pltpu.sync_copy(shared, out_n)

  return k(idx3, val3, zeros_hbm)


def _tc_loss_body(emb_ref, cp_ref, cn_ref, out_ref, acc_ref):
  i = pl.program_id(0)
  e = emb_ref[...]
  eb = e.astype(jnp.bfloat16)
  sqb = eb * eb
  ones8 = jnp.ones((8, DIM), jnp.bfloat16)
  n28 = lax.dot_general(ones8, sqb, (((1,), (1,)), ((), ())),
                        preferred_element_type=jnp.float32)
  n2l = n28[0:1]
  m = lax.rsqrt(jnp.maximum(n2l, EPS * EPS))
  cp = cp_ref[0]
  cn = cn_ref[0]
  w = jnp.concatenate(
      [cp, cn, cp * m, cn * m,
       jnp.zeros((4, BLK), jnp.float32)], axis=0)
  part = jnp.dot(w.astype(jnp.bfloat16), eb,
                 preferred_element_type=jnp.float32)

  @pl.when(i == 0)
  def _():
    acc_ref[...] = jnp.zeros_like(acc_ref)

  acc_ref[...] += part

  @pl.when(i == NUM_BLKS - 1)
  def _():
    a = acc_ref[...]
    n = jnp.float32(NUM_IDX)
    sp, sn, tp, tn = a[0], a[1], a[2], a[3]
    mcp = jnp.sum(sp * tp) / (n * jnp.maximum(jnp.sqrt(jnp.sum(sp * sp)),
                                              n * EPS))
    mcn = jnp.sum(sn * tn) / (n * jnp.maximum(jnp.sqrt(jnp.sum(sn * sn)),
                                              n * EPS))
    out_ref[...] = jnp.full((1, 1), 2.0 - mcp - mcn, jnp.float32)


def _tc_loss(embeddings, cp3, cn3):
  return pl.pallas_call(
      _tc_loss_body,
      grid=(NUM_BLKS,),
      in_specs=[
          pl.BlockSpec((BLK, DIM), lambda i: (i, 0)),
          pl.BlockSpec((1, 1, BLK), lambda i: (i, 0, 0)),
          pl.BlockSpec((1, 1, BLK), lambda i: (i, 0, 0)),
      ],
      out_specs=pl.BlockSpec((1, 1), lambda i: (0, 0)),
      out_shape=jax.ShapeDtypeStruct((1, 1), jnp.float32),
      scratch_shapes=[pltpu.VMEM((8, DIM), jnp.float32)],
  )(embeddings, cp3, cn3)


def kernel(embeddings, positive_nodes, negative_nodes):
  pad = PAD_IDX - NUM_IDX
  idx_p = jnp.concatenate(
      [positive_nodes.astype(jnp.int32),
       jnp.zeros((pad,), jnp.int32)]).reshape(SC_TILES, SC_CHUNKS, SC_LANEBLK)
  idx_n = jnp.concatenate(
      [negative_nodes.astype(jnp.int32),
       jnp.zeros((pad,), jnp.int32)]).reshape(SC_TILES, SC_CHUNKS, SC_LANEBLK)
  idx3 = jnp.stack([idx_p, idx_n], axis=0)
  val3 = jnp.concatenate(
      [jnp.ones((NUM_IDX,), jnp.float32),
       jnp.zeros((pad,), jnp.float32)]).reshape(SC_TILES, SC_CHUNKS,
                                                SC_LANEBLK)
  zeros_hbm = jnp.zeros((NUM_ROWS,), jnp.float32)

  hp, hn = _sc_histogram(idx3, val3, zeros_hbm)
  cp3 = hp.reshape(NUM_BLKS, 1, BLK)
  cn3 = hn.reshape(NUM_BLKS, 1, BLK)
  loss = _tc_loss(embeddings, cp3, cn3)
  return loss[0, 0]


# speedup vs baseline: 1.3682x; 1.0985x over previous
import functools

import jax
import jax.numpy as jnp
from jax import lax
from jax.experimental import pallas as pl
from jax.experimental.pallas import tpu as pltpu
from jax.experimental.pallas import tpu_sc as plsc

NUM_ROWS = 100000
DIM = 128
NUM_IDX = 50000
EPS = 1e-8

SC_TILES = 16
SC_CHUNKS = 25
SC_LANEBLK = 128
PAD_IDX = SC_TILES * SC_CHUNKS * SC_LANEBLK

BLK = 10000
NUM_BLKS = NUM_ROWS // BLK


def _sc_histogram(idx3, val3, zeros_hbm):
  mesh = plsc.VectorSubcoreMesh(core_axis_name="c", subcore_axis_name="s")

  @functools.partial(
      pl.kernel,
      mesh=mesh,
      out_type=[
          jax.ShapeDtypeStruct((NUM_ROWS,), jnp.float32),
          jax.ShapeDtypeStruct((NUM_ROWS,), jnp.float32),
      ],
      scratch_types=[
          pltpu.VMEM((SC_CHUNKS, SC_LANEBLK), jnp.int32),
          pltpu.VMEM((SC_CHUNKS, SC_LANEBLK), jnp.float32),
          pltpu.VMEM_SHARED((NUM_ROWS,), jnp.float32),
          pltpu.SemaphoreType.DMA,
      ],
  )
  def k(idx_hbm, val_hbm, zero_hbm, out_p, out_n, idx_v, val_v, shared, sem):
    c = lax.axis_index("c")
    s = lax.axis_index("s")

    @pl.when(s == 0)
    def _():
      pltpu.sync_copy(zero_hbm, shared)

    pltpu.sync_copy(idx_hbm.at[c, s], idx_v)
    pltpu.sync_copy(val_hbm.at[s], val_v)
    plsc.subcore_barrier()

    copies = [
        pltpu.async_copy(val_v.at[j], shared.at[idx_v.at[j]], sem, add=True)
        for j in range(SC_CHUNKS)
    ]
    for cpy in copies:
      cpy.wait()

    plsc.subcore_barrier()

    @pl.when((s == 0) & (c == 0))
    def _():
      pltpu.sync_copy(shared, out_p)

    @pl.when((s == 0) & (c == 1))
    def _():
      pltpu.sync_copy(shared, out_n)

  return k(idx3, val3, zeros_hbm)


def _tc_loss_body(emb_ref, cp_ref, cn_ref, out_ref, acc_ref):
  i = pl.program_id(0)
  e = emb_ref[...]
  eb = e.astype(jnp.bfloat16)
  sqb = eb * eb
  ones8 = jnp.ones((8, DIM), jnp.bfloat16)
  n28 = lax.dot_general(ones8, sqb, (((1,), (1,)), ((), ())),
                        preferred_element_type=jnp.float32)
  n2l = n28[0:1]
  m = lax.rsqrt(jnp.maximum(n2l, EPS * EPS))
  cp = cp_ref[0]
  cn = cn_ref[0]
  w = jnp.concatenate(
      [cp, cn, cp * m, cn * m,
       jnp.zeros((4, BLK), jnp.float32)], axis=0)
  part = jnp.dot(w.astype(jnp.bfloat16), eb,
                 preferred_element_type=jnp.float32)

  @pl.when(i == 0)
  def _():
    acc_ref[...] = jnp.zeros_like(acc_ref)

  acc_ref[...] += part

  @pl.when(i == NUM_BLKS - 1)
  def _():
    a = acc_ref[...]
    n = jnp.float32(NUM_IDX)
    sp, sn, tp, tn = a[0], a[1], a[2], a[3]
    mcp = jnp.sum(sp * tp) / (n * jnp.maximum(jnp.sqrt(jnp.sum(sp * sp)),
                                              n * EPS))
    mcn = jnp.sum(sn * tn) / (n * jnp.maximum(jnp.sqrt(jnp.sum(sn * sn)),
                                              n * EPS))
    out_ref[...] = jnp.full((1, 1), 2.0 - mcp - mcn, jnp.float32)


def _tc_loss(embeddings, cp3, cn3):
  return pl.pallas_call(
      _tc_loss_body,
      grid=(NUM_BLKS,),
      in_specs=[
          pl.BlockSpec((BLK, DIM), lambda i: (i, 0)),
          pl.BlockSpec((1, 1, BLK), lambda i: (i, 0, 0)),
          pl.BlockSpec((1, 1, BLK), lambda i: (i, 0, 0)),
      ],
      out_specs=pl.BlockSpec((1, 1), lambda i: (0, 0)),
      out_shape=jax.ShapeDtypeStruct((1, 1), jnp.float32),
      scratch_shapes=[pltpu.VMEM((8, DIM), jnp.float32)],
  )(embeddings, cp3, cn3)


def kernel(embeddings, positive_nodes, negative_nodes):
  pad = PAD_IDX - NUM_IDX
  idx_p = jnp.concatenate(
      [positive_nodes.astype(jnp.int32),
       jnp.zeros((pad,), jnp.int32)]).reshape(SC_TILES, SC_CHUNKS, SC_LANEBLK)
  idx_n = jnp.concatenate(
      [negative_nodes.astype(jnp.int32),
       jnp.zeros((pad,), jnp.int32)]).reshape(SC_TILES, SC_CHUNKS, SC_LANEBLK)
  idx3 = jnp.stack([idx_p, idx_n], axis=0)
  val3 = jnp.concatenate(
      [jnp.ones((NUM_IDX,), jnp.float32),
       jnp.zeros((pad,), jnp.float32)]).reshape(SC_TILES, SC_CHUNKS,
                                                SC_LANEBLK)
  zeros_hbm = jnp.zeros((NUM_ROWS,), jnp.float32)

  hp, hn = _sc_histogram(idx3, val3, zeros_hbm)
  cp3 = hp.reshape(NUM_BLKS, 1, BLK)
  cn3 = hn.reshape(NUM_BLKS, 1, BLK)
  loss = _tc_loss(embeddings, cp3, cn3)
  return loss[0, 0]


# speedup vs baseline: 1.4072x; 1.0285x over previous
import functools

import jax
import jax.numpy as jnp
from jax import lax
from jax.experimental import pallas as pl
from jax.experimental.pallas import tpu as pltpu
from jax.experimental.pallas import tpu_sc as plsc

NUM_ROWS = 100000
DIM = 128
NUM_IDX = 50000
EPS = 1e-8

SC_TILES = 16
SC_CHUNKS = 25
SC_LANEBLK = 128
PAD_IDX = SC_TILES * SC_CHUNKS * SC_LANEBLK

BLK = 20000
NUM_BLKS = NUM_ROWS // BLK


def _sc_histogram(idx3, val3, zeros_hbm):
  mesh = plsc.VectorSubcoreMesh(core_axis_name="c", subcore_axis_name="s")

  @functools.partial(
      pl.kernel,
      mesh=mesh,
      out_type=[
          jax.ShapeDtypeStruct((NUM_ROWS,), jnp.float32),
          jax.ShapeDtypeStruct((NUM_ROWS,), jnp.float32),
      ],
      scratch_types=[
          pltpu.VMEM((SC_CHUNKS, SC_LANEBLK), jnp.int32),
          pltpu.VMEM((SC_CHUNKS, SC_LANEBLK), jnp.float32),
          pltpu.VMEM_SHARED((NUM_ROWS,), jnp.float32),
          pltpu.SemaphoreType.DMA,
      ],
  )
  def k(idx_hbm, val_hbm, zero_hbm, out_p, out_n, idx_v, val_v, shared, sem):
    c = lax.axis_index("c")
    s = lax.axis_index("s")

    @pl.when(s == 0)
    def _():
      pltpu.sync_copy(zero_hbm, shared)

    pltpu.sync_copy(idx_hbm.at[c, s], idx_v)
    pltpu.sync_copy(val_hbm.at[s], val_v)
    plsc.subcore_barrier()

    copies = [
        pltpu.async_copy(val_v.at[j], shared.at[idx_v.at[j]], sem, add=True)
        for j in range(SC_CHUNKS)
    ]
    for cpy in copies:
      cpy.wait()

    plsc.subcore_barrier()

    @pl.when((s == 0) & (c == 0))
    def _():
      pltpu.sync_copy(shared, out_p)

    @pl.when((s == 0) & (c == 1))
    def _():
      pltpu.sync_copy(shared, out_n)

  return k(idx3, val3, zeros_hbm)


def _tc_loss_body(emb_ref, cp_ref, cn_ref, out_ref, acc_ref):
  i = pl.program_id(0)
  e = emb_ref[...]
  eb = e.astype(jnp.bfloat16)
  sqb = eb * eb
  ones8 = jnp.ones((8, DIM), jnp.bfloat16)
  n28 = lax.dot_general(ones8, sqb, (((1,), (1,)), ((), ())),
                        preferred_element_type=jnp.float32)
  n2l = n28[0:1]
  m = lax.rsqrt(jnp.maximum(n2l, EPS * EPS))
  cp = cp_ref[0]
  cn = cn_ref[0]
  w = jnp.concatenate(
      [cp, cn, cp * m, cn * m,
       jnp.zeros((4, BLK), jnp.float32)], axis=0)
  part = jnp.dot(w.astype(jnp.bfloat16), eb,
                 preferred_element_type=jnp.float32)

  @pl.when(i == 0)
  def _():
    acc_ref[...] = jnp.zeros_like(acc_ref)

  acc_ref[...] += part

  @pl.when(i == NUM_BLKS - 1)
  def _():
    a = acc_ref[...]
    n = jnp.float32(NUM_IDX)
    sp, sn, tp, tn = a[0], a[1], a[2], a[3]
    mcp = jnp.sum(sp * tp) / (n * jnp.maximum(jnp.sqrt(jnp.sum(sp * sp)),
                                              n * EPS))
    mcn = jnp.sum(sn * tn) / (n * jnp.maximum(jnp.sqrt(jnp.sum(sn * sn)),
                                              n * EPS))
    out_ref[...] = jnp.full((1, 1), 2.0 - mcp - mcn, jnp.float32)


def _tc_loss(embeddings, cp3, cn3):
  return pl.pallas_call(
      _tc_loss_body,
      grid=(NUM_BLKS,),
      in_specs=[
          pl.BlockSpec((BLK, DIM), lambda i: (i, 0)),
          pl.BlockSpec((1, 1, BLK), lambda i: (i, 0, 0)),
          pl.BlockSpec((1, 1, BLK), lambda i: (i, 0, 0)),
      ],
      out_specs=pl.BlockSpec((1, 1), lambda i: (0, 0)),
      out_shape=jax.ShapeDtypeStruct((1, 1), jnp.float32),
      scratch_shapes=[pltpu.VMEM((8, DIM), jnp.float32)],
  )(embeddings, cp3, cn3)


def kernel(embeddings, positive_nodes, negative_nodes):
  pad = PAD_IDX - NUM_IDX
  idx_p = jnp.concatenate(
      [positive_nodes.astype(jnp.int32),
       jnp.zeros((pad,), jnp.int32)]).reshape(SC_TILES, SC_CHUNKS, SC_LANEBLK)
  idx_n = jnp.concatenate(
      [negative_nodes.astype(jnp.int32),
       jnp.zeros((pad,), jnp.int32)]).reshape(SC_TILES, SC_CHUNKS, SC_LANEBLK)
  idx3 = jnp.stack([idx_p, idx_n], axis=0)
  val3 = jnp.concatenate(
      [jnp.ones((NUM_IDX,), jnp.float32),
       jnp.zeros((pad,), jnp.float32)]).reshape(SC_TILES, SC_CHUNKS,
                                                SC_LANEBLK)
  zeros_hbm = jnp.zeros((NUM_ROWS,), jnp.float32)

  hp, hn = _sc_histogram(idx3, val3, zeros_hbm)
  cp3 = hp.reshape(NUM_BLKS, 1, BLK)
  cn3 = hn.reshape(NUM_BLKS, 1, BLK)
  loss = _tc_loss(embeddings, cp3, cn3)
  return loss[0, 0]
